# Initial kernel scaffold; baseline (speedup 1.0000x reference)
import jax
import jax.numpy as jnp
from jax.experimental import pallas as pl


def kernel(x, y, edge_index, pair_idx):
    raise NotImplementedError("write your pallas kernel here")


# speedup vs baseline: 1.3395x; 1.3395x over previous
import functools

import jax
import jax.numpy as jnp
from jax import lax
from jax.experimental import pallas as pl
from jax.experimental.pallas import tpu as pltpu
from jax.experimental.pallas import tpu_sc as plsc

_LAMB = 0.7
_N = 10000
_D = 128
_LANES = 16

_NC = 2
_NS = 16
_NW = _NC * _NS
_PER_W = _N // _NW
_C = 104
_NCHUNK = _PER_W // _C
_TAIL = _N - _NW * _PER_W


def _mix_body(x_hbm, pair_hbm, out_hbm, idx_v, idx_t, a_v, b_v, sem):
    wid = lax.axis_index("s") * _NC + lax.axis_index("c")
    base = wid * _PER_W

    def do_chunk(gbase, sz, idx_ref):
        pltpu.sync_copy(pair_hbm.at[pl.ds(gbase, sz)], idx_ref)
        gather = pltpu.async_copy(x_hbm.at[idx_ref], b_v.at[pl.ds(0, sz)], sem)
        pltpu.sync_copy(x_hbm.at[pl.ds(gbase, sz)], a_v.at[pl.ds(0, sz)])
        gather.wait()

        def row(i, carry):
            for j in range(_D // _LANES):
                sl = pl.ds(j * _LANES, _LANES)
                a_v[i, sl] = a_v[i, sl] * _LAMB + b_v[i, sl] * (1.0 - _LAMB)
            return carry

        lax.fori_loop(0, sz, row, 0)
        pltpu.sync_copy(a_v.at[pl.ds(0, sz)], out_hbm.at[pl.ds(gbase, sz)])

    for k in range(_NCHUNK):
        do_chunk(base + k * _C, _C, idx_v)

    @pl.when(wid == _NW - 1)
    def _():
        do_chunk(_NW * _PER_W, _TAIL, idx_t)


@jax.jit
def _node_mixup_sc(x, pair_idx):
    mesh = plsc.VectorSubcoreMesh(core_axis_name="c", subcore_axis_name="s")
    call = pl.kernel(
        _mix_body,
        out_type=jax.ShapeDtypeStruct((_N, _D), jnp.float32),
        mesh=mesh,
        scratch_types=[
            pltpu.VMEM((_C,), jnp.int32),
            pltpu.VMEM((_TAIL,), jnp.int32),
            pltpu.VMEM((_C, _D), jnp.float32),
            pltpu.VMEM((_C, _D), jnp.float32),
            pltpu.SemaphoreType.DMA,
        ],
    )
    return call(x, pair_idx)


def kernel(x, y, edge_index, pair_idx):
    x_mix = _node_mixup_sc(x, pair_idx)
    new_y = y.astype(jnp.int32)
    return (x_mix, new_y, edge_index)


# speedup vs baseline: 1.3991x; 1.0445x over previous
import jax
import jax.numpy as jnp
from jax import lax
from jax.experimental import pallas as pl
from jax.experimental.pallas import tpu as pltpu
from jax.experimental.pallas import tpu_sc as plsc

_LAMB = 0.7
_N = 10000
_D = 128
_LANES = 16

_NC = 2
_NS = 16
_NW = _NC * _NS
_PER_W = _N // _NW
_C = 104
_NCHUNK = _PER_W // _C
_TAIL = _N - _NW * _PER_W


def _mix_rows(a_v, b_v, lo, hi):
    @plsc.parallel_loop(lo, hi, unroll=4)
    def _(i):
        for j in range(_D // _LANES):
            sl = pl.ds(j * _LANES, _LANES)
            a_v[i, sl] = a_v[i, sl] * _LAMB + b_v[i, sl] * (1.0 - _LAMB)


def _mix_body(x_hbm, pair_hbm, out_hbm, idx_v, idx_t, a_v, b_v, at_v, bt_v,
              gsem0, gsem1, gsem2, lsem, osem):
    wid = lax.axis_index("s") * _NC + lax.axis_index("c")
    base = wid * _PER_W
    gsems = (gsem0, gsem1, gsem2)

    pltpu.sync_copy(pair_hbm.at[pl.ds(base, _PER_W)], idx_v)
    gathers = [
        pltpu.async_copy(x_hbm.at[idx_v.at[pl.ds(k * _C, _C)]],
                         b_v.at[pl.ds(k * _C, _C)], gsems[k])
        for k in range(_NCHUNK)
    ]
    pltpu.async_copy(x_hbm.at[pl.ds(base, _PER_W)], a_v, lsem).wait()

    stores = []
    for k in range(_NCHUNK):
        gathers[k].wait()
        _mix_rows(a_v, b_v, k * _C, (k + 1) * _C)
        stores.append(pltpu.async_copy(
            a_v.at[pl.ds(k * _C, _C)],
            out_hbm.at[pl.ds(base + k * _C, _C)], osem))

    @pl.when(wid == _NW - 1)
    def _():
        tbase = _NW * _PER_W
        pltpu.sync_copy(pair_hbm.at[pl.ds(tbase, _TAIL)], idx_t)
        pltpu.async_copy(x_hbm.at[idx_t], bt_v, gsem0).wait()
        pltpu.sync_copy(x_hbm.at[pl.ds(tbase, _TAIL)], at_v)
        _mix_rows(at_v, bt_v, 0, _TAIL)
        pltpu.sync_copy(at_v, out_hbm.at[pl.ds(tbase, _TAIL)])

    for s in stores:
        s.wait()


@jax.jit
def _node_mixup_sc(x, pair_idx):
    mesh = plsc.VectorSubcoreMesh(core_axis_name="c", subcore_axis_name="s")
    call = pl.kernel(
        _mix_body,
        out_type=jax.ShapeDtypeStruct((_N, _D), jnp.float32),
        mesh=mesh,
        scratch_types=[
            pltpu.VMEM((_PER_W,), jnp.int32),
            pltpu.VMEM((_TAIL,), jnp.int32),
            pltpu.VMEM((_PER_W, _D), jnp.float32),
            pltpu.VMEM((_PER_W, _D), jnp.float32),
            pltpu.VMEM((_TAIL, _D), jnp.float32),
            pltpu.VMEM((_TAIL, _D), jnp.float32),
            pltpu.SemaphoreType.DMA,
            pltpu.SemaphoreType.DMA,
            pltpu.SemaphoreType.DMA,
            pltpu.SemaphoreType.DMA,
            pltpu.SemaphoreType.DMA,
        ],
    )
    return call(x, pair_idx)


def kernel(x, y, edge_index, pair_idx):
    x_mix = _node_mixup_sc(x, pair_idx)
    new_y = y.astype(jnp.int32)
    return (x_mix, new_y, edge_index)
